# baseline (speedup 1.0000x reference)
import functools

import jax
import jax.numpy as jnp
from jax import lax
from jax.experimental import pallas as pl
from jax.experimental.pallas import tpu as pltpu, tpu_sc as plsc

B = 8
BH = 4
P0 = 16384
P1 = 1024
S = 16
F_IN = 16
F_ENC = 128
LATENT = 256
F_DEC0 = 128
F_OUT = 16

NW = 32
_STREAM = 128
_PW = P0 // NW
_PCH = 128
_NCH = _PW // _PCH
_GROWS = _PCH * S
_FIRES = _GROWS // _STREAM


def _sc_mesh():
    return plsc.VectorSubcoreMesh(core_axis_name="c", subcore_axis_name="s")


def _worker_id():
    return lax.axis_index("s") * 2 + lax.axis_index("c")


def _make_enc_gather(b0):
    @functools.partial(
        pl.kernel,
        mesh=_sc_mesh(),
        out_type=jax.ShapeDtypeStruct((BH * P0 * S, F_IN), jnp.float32),
        compiler_params=pltpu.CompilerParams(use_tc_tiling_on_sc=False),
        scratch_types=[
            pltpu.VMEM((_GROWS,), jnp.int32),
            pltpu.VMEM((_GROWS, F_IN), jnp.float32),
            pltpu.SemaphoreType.DMA,
        ],
    )
    def enc_gather(x_hbm, sp_hbm, out_hbm, sp_v, rows_v, sem):
        wid = _worker_id()

        def chunk_body(pc, carry):
            pbase = wid * _PW + pc * _PCH
            pltpu.sync_copy(sp_hbm.at[pl.ds(pbase * S, _GROWS)], sp_v)
            for b in range(BH):
                copies = []
                for j in range(_FIRES):
                    copies.append(pltpu.async_copy(
                        x_hbm.at[b0 + b].at[
                            sp_v.at[pl.ds(j * _STREAM, _STREAM)]],
                        rows_v.at[pl.ds(j * _STREAM, _STREAM)],
                        sem))
                for c in copies:
                    c.wait()
                pltpu.sync_copy(
                    rows_v, out_hbm.at[pl.ds((b * P0 + pbase) * S, _GROWS)])
            return carry

        lax.fori_loop(0, _NCH, chunk_body, 0)

    return enc_gather


_enc_gather_halves = (_make_enc_gather(0), _make_enc_gather(BH))


@functools.partial(
    pl.kernel,
    mesh=_sc_mesh(),
    out_type=jax.ShapeDtypeStruct((BH, P0, F_OUT), jnp.float32),
    compiler_params=pltpu.CompilerParams(use_tc_tiling_on_sc=False),
    scratch_types=[
        pltpu.VMEM((_GROWS,), jnp.int32),
        pltpu.VMEM((_GROWS, F_OUT), jnp.float32),
        pltpu.VMEM((_PCH, F_OUT), jnp.float32),
        pltpu.VMEM((F_OUT,), jnp.float32),
        pltpu.SemaphoreType.DMA,
    ],
)
def _sc_dec_bag(y_hbm, sp_hbm, bdec_hbm, out_hbm,
                idx_v, rows_v, out_v, bias_v, sem):
    wid = _worker_id()
    pltpu.sync_copy(bdec_hbm, bias_v)
    iot = lax.iota(jnp.int32, 16)

    def chunk_body(pc, carry):
        pbase = wid * _PW + pc * _PCH
        pltpu.sync_copy(sp_hbm.at[pl.ds(pbase * S, _GROWS)], idx_v)

        def mk_idx(r, rcarry):
            sl = pl.ds(r * S, S)
            idx_v[sl] = idx_v[sl] * S + iot
            return rcarry

        lax.fori_loop(0, _PCH, mk_idx, 0)
        bias = bias_v[...]
        for b in range(BH):
            if b > 0:
                def bump(r, rcarry):
                    sl = pl.ds(r * S, S)
                    idx_v[sl] = idx_v[sl] + (P0 * S)
                    return rcarry
                lax.fori_loop(0, _PCH, bump, 0)
            copies = []
            for j in range(_FIRES):
                copies.append(pltpu.async_copy(
                    y_hbm.at[idx_v.at[pl.ds(j * _STREAM, _STREAM)]],
                    rows_v.at[pl.ds(j * _STREAM, _STREAM)],
                    sem))
            for c in copies:
                c.wait()

            def row_body(r, rcarry):
                acc = bias
                for s in range(S):
                    acc = acc + rows_v[r * S + s]
                keep = jnp.where(pbase + r == P0 - 1,
                                 jnp.float32(0.0), jnp.float32(1.0))
                out_v[r] = acc * keep
                return rcarry

            lax.fori_loop(0, _PCH, row_body, 0)
            pltpu.sync_copy(out_v, out_hbm.at[b].at[pl.ds(pbase, _PCH)])
        return carry

    lax.fori_loop(0, _NCH, chunk_body, 0)


_TKA = 2048
_KBA = P0 // _TKA


def _tc_a_body(g_ref, w_ref, b_ref, d_ref, o_ref, dc_ref):
    k = pl.program_id(0)
    b = pl.program_id(1)

    @pl.when(b == 0)
    def _():
        dc_ref[...] = d_ref[...].astype(jnp.bfloat16)

    g = g_ref[0].astype(jnp.bfloat16)
    h = lax.dot_general(g, w_ref[...].astype(jnp.bfloat16),
                        (((1,), (1,)), ((), ())),
                        preferred_element_type=jnp.float32)
    h = h + b_ref[...]
    h = jnp.where(h > 0, h, jnp.exp(jnp.minimum(h, 0.0)) - 1.0)
    c = lax.dot_general(dc_ref[...], h.astype(jnp.bfloat16),
                        (((1,), (0,)), ((), ())),
                        preferred_element_type=jnp.float32)

    @pl.when(k == 0)
    def _():
        o_ref[b] = c

    @pl.when(k != 0)
    def _():
        o_ref[b] = o_ref[b] + c


def _tc_a(g0r, w_enc, b_enc2, d0m):
    nb = g0r.shape[0]
    return pl.pallas_call(
        _tc_a_body,
        grid=(_KBA, nb),
        in_specs=[
            pl.BlockSpec((1, _TKA, S * F_IN), lambda k, b: (b, k, 0)),
            pl.BlockSpec((F_ENC, S * F_IN), lambda k, b: (0, 0)),
            pl.BlockSpec((1, F_ENC), lambda k, b: (0, 0)),
            pl.BlockSpec((P1, _TKA), lambda k, b: (0, k)),
        ],
        out_specs=pl.BlockSpec((nb, P1, F_ENC), lambda k, b: (0, 0, 0)),
        out_shape=jax.ShapeDtypeStruct((nb, P1, F_ENC), jnp.float32),
        scratch_shapes=[pltpu.VMEM((P1, _TKA), jnp.bfloat16)],
    )(g0r, w_enc, b_enc2, d0m)


_CKB = 8192
_KBB = (P1 * F_ENC) // _CKB


def _tc_b1_body(ha_ref, hb_ref, w_ref, b_ref, o_ref):
    k = pl.program_id(0)
    h = jnp.concatenate([ha_ref[...], hb_ref[...]], axis=0)
    c = lax.dot_general(h, w_ref[...], (((1,), (1,)), ((), ())),
                        preferred_element_type=jnp.float32)

    @pl.when(k == 0)
    def _():
        o_ref[...] = c + b_ref[...]

    @pl.when(k != 0)
    def _():
        o_ref[...] = o_ref[...] + c


def _tc_b1(hda, hdb, w_fc_enc, b_fc_enc2):
    return pl.pallas_call(
        _tc_b1_body,
        grid=(_KBB,),
        in_specs=[
            pl.BlockSpec((BH, _CKB), lambda k: (0, k)),
            pl.BlockSpec((BH, _CKB), lambda k: (0, k)),
            pl.BlockSpec((LATENT, _CKB), lambda k: (0, k)),
            pl.BlockSpec((1, LATENT), lambda k: (0, 0)),
        ],
        out_specs=pl.BlockSpec((B, LATENT), lambda k: (0, 0)),
        out_shape=jax.ShapeDtypeStruct((B, LATENT), jnp.float32),
    )(hda, hdb, w_fc_enc, b_fc_enc2)


_CNB = 8192
_NBB = (P1 * F_DEC0) // _CNB


def _tc_b2_body(z_ref, w_ref, b_ref, o_ref):
    c = lax.dot_general(z_ref[...], w_ref[...], (((1,), (1,)), ((), ())),
                        preferred_element_type=jnp.float32)
    o_ref[...] = c + b_ref[...]


def _tc_b2(z, w_fc_dec, b_fc_dec2):
    return pl.pallas_call(
        _tc_b2_body,
        grid=(_NBB,),
        in_specs=[
            pl.BlockSpec((B, LATENT), lambda n: (0, 0)),
            pl.BlockSpec((_CNB, LATENT), lambda n: (n, 0)),
            pl.BlockSpec((1, _CNB), lambda n: (0, n)),
        ],
        out_specs=pl.BlockSpec((B, _CNB), lambda n: (0, n)),
        out_shape=jax.ShapeDtypeStruct((B, P1 * F_DEC0), jnp.float32),
    )(z, w_fc_dec, b_fc_dec2)


_TMC = 2048
_MBC = P0 // _TMC


def _tc_c_body(u_ref, d_ref, w_ref, o_ref, uc_ref):
    b = pl.program_id(1)

    @pl.when(b == 0)
    def _():
        uc_ref[...] = u_ref[...].astype(jnp.bfloat16)

    u = lax.dot_general(uc_ref[...], d_ref[0].astype(jnp.bfloat16),
                        (((1,), (0,)), ((), ())),
                        preferred_element_type=jnp.float32)
    y = lax.dot_general(u.astype(jnp.bfloat16),
                        w_ref[...].astype(jnp.bfloat16),
                        (((1,), (0,)), ((), ())),
                        preferred_element_type=jnp.float32)
    o_ref[0] = y


def _tc_c(u0, dd3, wd2):
    nb = dd3.shape[0]
    return pl.pallas_call(
        _tc_c_body,
        grid=(_MBC, nb),
        in_specs=[
            pl.BlockSpec((_TMC, P1), lambda m, b: (m, 0)),
            pl.BlockSpec((1, P1, F_DEC0), lambda m, b: (b, 0, 0)),
            pl.BlockSpec((F_DEC0, S * F_OUT), lambda m, b: (0, 0)),
        ],
        out_specs=pl.BlockSpec((1, _TMC, S * F_OUT), lambda m, b: (b, m, 0)),
        out_shape=jax.ShapeDtypeStruct((nb, P0, S * F_OUT), jnp.float32),
        scratch_shapes=[pltpu.VMEM((_TMC, P1), jnp.bfloat16)],
    )(u0, dd3, wd2)


def kernel(x, spirals0, W_enc, b_enc, D0, W_fc_enc, b_fc_enc, W_fc_dec,
           b_fc_dec, U0, W_dec, b_dec):
    sp_flat = spirals0.astype(jnp.int32).reshape(-1)

    d0m = D0.at[:, P0 - 1].set(0.0)
    wd2 = W_dec.reshape(F_OUT, S, F_DEC0).transpose(2, 1, 0).reshape(
        F_DEC0, S * F_OUT)
    b_enc2 = b_enc.reshape(1, F_ENC)

    g0a = _enc_gather_halves[0](x, sp_flat)
    g0b = _enc_gather_halves[1](x, sp_flat)
    hda = _tc_a(g0a.reshape(BH, P0, S * F_IN), W_enc, b_enc2, d0m)
    hdb = _tc_a(g0b.reshape(BH, P0, S * F_IN), W_enc, b_enc2, d0m)

    z = _tc_b1(hda.reshape(BH, P1 * F_ENC), hdb.reshape(BH, P1 * F_ENC),
               W_fc_enc, b_fc_enc.reshape(1, LATENT))
    dd = _tc_b2(z, W_fc_dec, b_fc_dec.reshape(1, P1 * F_DEC0))
    dd3 = dd.reshape(B, P1, F_DEC0)

    ya = _tc_c(U0, dd3[:BH], wd2)
    yb = _tc_c(U0, dd3[BH:], wd2)
    outa = _sc_dec_bag(ya.reshape(BH * P0 * S, F_OUT), sp_flat, b_dec)
    outb = _sc_dec_bag(yb.reshape(BH * P0 * S, F_OUT), sp_flat, b_dec)
    return jnp.concatenate([outa, outb], axis=0)


# speedup vs baseline: 1.3287x; 1.3287x over previous
import functools

import jax
import jax.numpy as jnp
from jax import lax
from jax.experimental import pallas as pl
from jax.experimental.pallas import tpu as pltpu, tpu_sc as plsc

B = 8
BH = 4
P0 = 16384
P1 = 1024
S = 16
SH = 8
F_IN = 16
F_ENC = 128
LATENT = 256
F_DEC0 = 128
F_OUT = 16

NW = 32
_STREAM = 128
_PW = P0 // NW
_PCH = 128
_NCH = _PW // _PCH
_HROWS = _PCH * SH
_FIRES = _HROWS // _STREAM


def _sc_mesh():
    return plsc.VectorSubcoreMesh(core_axis_name="c", subcore_axis_name="s")


def _worker_id():
    return lax.axis_index("s") * 2 + lax.axis_index("c")


def _make_enc_gather(b0):
    @functools.partial(
        pl.kernel,
        mesh=_sc_mesh(),
        out_type=(jax.ShapeDtypeStruct((BH * P0 * SH, F_IN), jnp.float32),
                  jax.ShapeDtypeStruct((BH * P0 * SH, F_IN), jnp.float32)),
        compiler_params=pltpu.CompilerParams(use_tc_tiling_on_sc=False),
        scratch_types=[
            pltpu.VMEM((_HROWS,), jnp.int32),
            pltpu.VMEM((_HROWS,), jnp.int32),
            pltpu.VMEM((_HROWS, F_IN), jnp.float32),
            pltpu.VMEM((_HROWS, F_IN), jnp.float32),
            pltpu.SemaphoreType.DMA,
        ],
    )
    def enc_gather(x_hbm, splo_hbm, sphi_hbm, glo_hbm, ghi_hbm,
                   splo_v, sphi_v, rlo_v, rhi_v, sem):
        wid = _worker_id()

        def chunk_body(pc, carry):
            pbase = wid * _PW + pc * _PCH
            pltpu.sync_copy(splo_hbm.at[pl.ds(pbase * SH, _HROWS)], splo_v)
            pltpu.sync_copy(sphi_hbm.at[pl.ds(pbase * SH, _HROWS)], sphi_v)
            for b in range(BH):
                copies = []
                for j in range(_FIRES):
                    sl = pl.ds(j * _STREAM, _STREAM)
                    copies.append(pltpu.async_copy(
                        x_hbm.at[b0 + b].at[splo_v.at[sl]], rlo_v.at[sl],
                        sem))
                    copies.append(pltpu.async_copy(
                        x_hbm.at[b0 + b].at[sphi_v.at[sl]], rhi_v.at[sl],
                        sem))
                for c in copies:
                    c.wait()
                dst = pl.ds((b * P0 + pbase) * SH, _HROWS)
                pltpu.sync_copy(rlo_v, glo_hbm.at[dst])
                pltpu.sync_copy(rhi_v, ghi_hbm.at[dst])
            return carry

        lax.fori_loop(0, _NCH, chunk_body, 0)

    return enc_gather


_enc_gather_halves = (_make_enc_gather(0), _make_enc_gather(BH))


@functools.partial(
    pl.kernel,
    mesh=_sc_mesh(),
    out_type=jax.ShapeDtypeStruct((BH, P0, F_OUT), jnp.float32),
    compiler_params=pltpu.CompilerParams(use_tc_tiling_on_sc=False),
    scratch_types=[
        pltpu.VMEM((_HROWS,), jnp.int32),
        pltpu.VMEM((_HROWS,), jnp.int32),
        pltpu.VMEM((_HROWS, F_OUT), jnp.float32),
        pltpu.VMEM((_HROWS, F_OUT), jnp.float32),
        pltpu.VMEM((_PCH, F_OUT), jnp.float32),
        pltpu.VMEM((F_OUT,), jnp.float32),
        pltpu.SemaphoreType.DMA,
    ],
)
def _sc_dec_bag(ylo_hbm, yhi_hbm, splo_hbm, sphi_hbm, bdec_hbm, out_hbm,
                ilo_v, ihi_v, rlo_v, rhi_v, out_v, bias_v, sem):
    wid = _worker_id()
    pltpu.sync_copy(bdec_hbm, bias_v)
    lane8 = jnp.bitwise_and(lax.iota(jnp.int32, 16), 7)

    def chunk_body(pc, carry):
        pbase = wid * _PW + pc * _PCH
        pltpu.sync_copy(splo_hbm.at[pl.ds(pbase * SH, _HROWS)], ilo_v)
        pltpu.sync_copy(sphi_hbm.at[pl.ds(pbase * SH, _HROWS)], ihi_v)

        def mk_idx(r, rcarry):
            sl = pl.ds(r * 16, 16)
            ilo_v[sl] = ilo_v[sl] * SH + lane8
            ihi_v[sl] = ihi_v[sl] * SH + lane8
            return rcarry

        lax.fori_loop(0, _HROWS // 16, mk_idx, 0)
        bias = bias_v[...]
        for b in range(BH):
            if b > 0:
                def bump(r, rcarry):
                    sl = pl.ds(r * 16, 16)
                    ilo_v[sl] = ilo_v[sl] + (P0 * SH)
                    ihi_v[sl] = ihi_v[sl] + (P0 * SH)
                    return rcarry
                lax.fori_loop(0, _HROWS // 16, bump, 0)
            copies = []
            for j in range(_FIRES):
                sl = pl.ds(j * _STREAM, _STREAM)
                copies.append(pltpu.async_copy(
                    ylo_hbm.at[ilo_v.at[sl]], rlo_v.at[sl], sem))
                copies.append(pltpu.async_copy(
                    yhi_hbm.at[ihi_v.at[sl]], rhi_v.at[sl], sem))
            for c in copies:
                c.wait()

            def row_body(r, rcarry):
                acc = bias
                for j in range(SH):
                    acc = acc + rlo_v[r * SH + j]
                for j in range(SH):
                    acc = acc + rhi_v[r * SH + j]
                keep = jnp.where(pbase + r == P0 - 1,
                                 jnp.float32(0.0), jnp.float32(1.0))
                out_v[r] = acc * keep
                return rcarry

            lax.fori_loop(0, _PCH, row_body, 0)
            pltpu.sync_copy(out_v, out_hbm.at[b].at[pl.ds(pbase, _PCH)])
        return carry

    lax.fori_loop(0, _NCH, chunk_body, 0)


_TKA = 2048
_KBA = P0 // _TKA


def _tc_a_body(glo_ref, ghi_ref, wlo_ref, whi_ref, b_ref, d_ref, o_ref,
               dc_ref):
    k = pl.program_id(0)
    b = pl.program_id(1)

    @pl.when(b == 0)
    def _():
        dc_ref[...] = d_ref[...].astype(jnp.bfloat16)

    h = lax.dot_general(glo_ref[...].astype(jnp.bfloat16),
                        wlo_ref[...].astype(jnp.bfloat16),
                        (((1,), (1,)), ((), ())),
                        preferred_element_type=jnp.float32)
    h = h + lax.dot_general(ghi_ref[...].astype(jnp.bfloat16),
                            whi_ref[...].astype(jnp.bfloat16),
                            (((1,), (1,)), ((), ())),
                            preferred_element_type=jnp.float32)
    h = h + b_ref[...]
    h = jnp.where(h > 0, h, jnp.exp(jnp.minimum(h, 0.0)) - 1.0)
    c = lax.dot_general(dc_ref[...], h.astype(jnp.bfloat16),
                        (((1,), (0,)), ((), ())),
                        preferred_element_type=jnp.float32)

    @pl.when(k == 0)
    def _():
        o_ref[b] = c

    @pl.when(k != 0)
    def _():
        o_ref[b] = o_ref[b] + c


def _tc_a(glo, ghi, w_lo, w_hi, b_enc2, d0m, nb):
    def rows(k, b):
        return (b * _KBA + k, 0)

    return pl.pallas_call(
        _tc_a_body,
        grid=(_KBA, nb),
        in_specs=[
            pl.BlockSpec((_TKA, SH * F_IN), rows),
            pl.BlockSpec((_TKA, SH * F_IN), rows),
            pl.BlockSpec((F_ENC, SH * F_IN), lambda k, b: (0, 0)),
            pl.BlockSpec((F_ENC, SH * F_IN), lambda k, b: (0, 0)),
            pl.BlockSpec((1, F_ENC), lambda k, b: (0, 0)),
            pl.BlockSpec((P1, _TKA), lambda k, b: (0, k)),
        ],
        out_specs=pl.BlockSpec((nb, P1, F_ENC), lambda k, b: (0, 0, 0)),
        out_shape=jax.ShapeDtypeStruct((nb, P1, F_ENC), jnp.float32),
        scratch_shapes=[pltpu.VMEM((P1, _TKA), jnp.bfloat16)],
    )(glo, ghi, w_lo, w_hi, b_enc2, d0m)


_CKB = 8192
_KBB = (P1 * F_ENC) // _CKB


def _tc_b1_body(ha_ref, hb_ref, w_ref, b_ref, o_ref):
    k = pl.program_id(0)
    h = jnp.concatenate([ha_ref[...], hb_ref[...]], axis=0)
    c = lax.dot_general(h, w_ref[...], (((1,), (1,)), ((), ())),
                        preferred_element_type=jnp.float32)

    @pl.when(k == 0)
    def _():
        o_ref[...] = c + b_ref[...]

    @pl.when(k != 0)
    def _():
        o_ref[...] = o_ref[...] + c


def _tc_b1(hda, hdb, w_fc_enc, b_fc_enc2):
    return pl.pallas_call(
        _tc_b1_body,
        grid=(_KBB,),
        in_specs=[
            pl.BlockSpec((BH, _CKB), lambda k: (0, k)),
            pl.BlockSpec((BH, _CKB), lambda k: (0, k)),
            pl.BlockSpec((LATENT, _CKB), lambda k: (0, k)),
            pl.BlockSpec((1, LATENT), lambda k: (0, 0)),
        ],
        out_specs=pl.BlockSpec((B, LATENT), lambda k: (0, 0)),
        out_shape=jax.ShapeDtypeStruct((B, LATENT), jnp.float32),
    )(hda, hdb, w_fc_enc, b_fc_enc2)


_CNB = 8192
_NBB = (P1 * F_DEC0) // _CNB


def _tc_b2_body(z_ref, w_ref, b_ref, o_ref):
    c = lax.dot_general(z_ref[...], w_ref[...], (((1,), (1,)), ((), ())),
                        preferred_element_type=jnp.float32)
    o_ref[...] = c + b_ref[...]


def _tc_b2(z, w_fc_dec, b_fc_dec2):
    return pl.pallas_call(
        _tc_b2_body,
        grid=(_NBB,),
        in_specs=[
            pl.BlockSpec((B, LATENT), lambda n: (0, 0)),
            pl.BlockSpec((_CNB, LATENT), lambda n: (n, 0)),
            pl.BlockSpec((1, _CNB), lambda n: (0, n)),
        ],
        out_specs=pl.BlockSpec((B, _CNB), lambda n: (0, n)),
        out_shape=jax.ShapeDtypeStruct((B, P1 * F_DEC0), jnp.float32),
    )(z, w_fc_dec, b_fc_dec2)


_TMC = 2048
_MBC = P0 // _TMC


def _tc_c_body(u_ref, d_ref, wlo_ref, whi_ref, olo_ref, ohi_ref, uc_ref):
    b = pl.program_id(1)

    @pl.when(b == 0)
    def _():
        uc_ref[...] = u_ref[...].astype(jnp.bfloat16)

    u = lax.dot_general(uc_ref[...], d_ref[0].astype(jnp.bfloat16),
                        (((1,), (0,)), ((), ())),
                        preferred_element_type=jnp.float32)
    ub = u.astype(jnp.bfloat16)
    olo_ref[...] = lax.dot_general(ub, wlo_ref[...].astype(jnp.bfloat16),
                                   (((1,), (0,)), ((), ())),
                                   preferred_element_type=jnp.float32)
    ohi_ref[...] = lax.dot_general(ub, whi_ref[...].astype(jnp.bfloat16),
                                   (((1,), (0,)), ((), ())),
                                   preferred_element_type=jnp.float32)


def _tc_c(u0, dd3, wd2lo, wd2hi):
    nb = dd3.shape[0]

    def rows(m, b):
        return (b * _MBC + m, 0)

    return pl.pallas_call(
        _tc_c_body,
        grid=(_MBC, nb),
        in_specs=[
            pl.BlockSpec((_TMC, P1), lambda m, b: (m, 0)),
            pl.BlockSpec((1, P1, F_DEC0), lambda m, b: (b, 0, 0)),
            pl.BlockSpec((F_DEC0, SH * F_OUT), lambda m, b: (0, 0)),
            pl.BlockSpec((F_DEC0, SH * F_OUT), lambda m, b: (0, 0)),
        ],
        out_specs=(pl.BlockSpec((_TMC, SH * F_OUT), rows),
                   pl.BlockSpec((_TMC, SH * F_OUT), rows)),
        out_shape=(jax.ShapeDtypeStruct((nb * P0, SH * F_OUT), jnp.float32),
                   jax.ShapeDtypeStruct((nb * P0, SH * F_OUT), jnp.float32)),
        scratch_shapes=[pltpu.VMEM((_TMC, P1), jnp.bfloat16)],
    )(u0, dd3, wd2lo, wd2hi)


def kernel(x, spirals0, W_enc, b_enc, D0, W_fc_enc, b_fc_enc, W_fc_dec,
           b_fc_dec, U0, W_dec, b_dec):
    sp = spirals0.astype(jnp.int32)
    sp_lo = sp[:, :SH].reshape(-1)
    sp_hi = sp[:, SH:].reshape(-1)

    d0m = D0.at[:, P0 - 1].set(0.0)
    wd2 = W_dec.reshape(F_OUT, S, F_DEC0).transpose(2, 1, 0).reshape(
        F_DEC0, S * F_OUT)
    wd2lo, wd2hi = wd2[:, :SH * F_OUT], wd2[:, SH * F_OUT:]
    w_lo, w_hi = W_enc[:, :SH * F_IN], W_enc[:, SH * F_IN:]
    b_enc2 = b_enc.reshape(1, F_ENC)

    gla, gha = _enc_gather_halves[0](x, sp_lo, sp_hi)
    glb, ghb = _enc_gather_halves[1](x, sp_lo, sp_hi)
    hda = _tc_a(gla.reshape(BH * P0, SH * F_IN),
                gha.reshape(BH * P0, SH * F_IN), w_lo, w_hi, b_enc2, d0m, BH)
    hdb = _tc_a(glb.reshape(BH * P0, SH * F_IN),
                ghb.reshape(BH * P0, SH * F_IN), w_lo, w_hi, b_enc2, d0m, BH)

    z = _tc_b1(hda.reshape(BH, P1 * F_ENC), hdb.reshape(BH, P1 * F_ENC),
               W_fc_enc, b_fc_enc.reshape(1, LATENT))
    dd = _tc_b2(z, W_fc_dec, b_fc_dec.reshape(1, P1 * F_DEC0))
    dd3 = dd.reshape(B, P1, F_DEC0)

    yla, yha = _tc_c(U0, dd3[:BH], wd2lo, wd2hi)
    ylb, yhb = _tc_c(U0, dd3[BH:], wd2lo, wd2hi)
    outa = _sc_dec_bag(yla.reshape(BH * P0 * SH, F_OUT),
                       yha.reshape(BH * P0 * SH, F_OUT), sp_lo, sp_hi, b_dec)
    outb = _sc_dec_bag(ylb.reshape(BH * P0 * SH, F_OUT),
                       yhb.reshape(BH * P0 * SH, F_OUT), sp_lo, sp_hi, b_dec)
    return jnp.concatenate([outa, outb], axis=0)


# speedup vs baseline: 1.3792x; 1.0380x over previous
import functools

import jax
import jax.numpy as jnp
from jax import lax
from jax.experimental import pallas as pl
from jax.experimental.pallas import tpu as pltpu, tpu_sc as plsc

B = 8
BH = 4
P0 = 16384
P1 = 1024
S = 16
SH = 8
F_IN = 16
F_ENC = 128
LATENT = 256
F_DEC0 = 128
F_OUT = 16

NW = 32
_STREAM = 128
_PW = P0 // NW
_PCH = 128
_NCH = _PW // _PCH
_HROWS = _PCH * SH
_FIRES = _HROWS // _STREAM


def _sc_mesh():
    return plsc.VectorSubcoreMesh(core_axis_name="c", subcore_axis_name="s")


def _worker_id():
    return lax.axis_index("s") * 2 + lax.axis_index("c")


def _make_enc_gather(b0):
    @functools.partial(
        pl.kernel,
        mesh=_sc_mesh(),
        out_type=(jax.ShapeDtypeStruct((BH * P0 * SH, F_IN), jnp.float32),
                  jax.ShapeDtypeStruct((BH * P0 * SH, F_IN), jnp.float32)),
        compiler_params=pltpu.CompilerParams(use_tc_tiling_on_sc=False),
        scratch_types=[
            pltpu.VMEM((_HROWS,), jnp.int32),
            pltpu.VMEM((_HROWS,), jnp.int32),
            pltpu.VMEM((_HROWS, F_IN), jnp.float32),
            pltpu.VMEM((_HROWS, F_IN), jnp.float32),
            pltpu.SemaphoreType.DMA,
        ],
    )
    def enc_gather(x_hbm, splo_hbm, sphi_hbm, glo_hbm, ghi_hbm,
                   splo_v, sphi_v, rlo_v, rhi_v, sem):
        wid = _worker_id()
        nvec = _HROWS // 16

        def chunk_body(pc, carry):
            pbase = wid * _PW + pc * _PCH
            pltpu.sync_copy(splo_hbm.at[pl.ds(pbase * SH, _HROWS)], splo_v)
            pltpu.sync_copy(sphi_hbm.at[pl.ds(pbase * SH, _HROWS)], sphi_v)
            for b in range(BH):
                off = (b0 if b == 0 else 0) * P0 + (P0 if b > 0 else 0)
                if off:
                    def bump(r, rcarry):
                        sl = pl.ds(r * 16, 16)
                        splo_v[sl] = splo_v[sl] + off
                        sphi_v[sl] = sphi_v[sl] + off
                        return rcarry
                    lax.fori_loop(0, nvec, bump, 0)
                copies = []
                for j in range(_FIRES):
                    sl = pl.ds(j * _STREAM, _STREAM)
                    copies.append(pltpu.async_copy(
                        x_hbm.at[splo_v.at[sl]], rlo_v.at[sl], sem))
                    copies.append(pltpu.async_copy(
                        x_hbm.at[sphi_v.at[sl]], rhi_v.at[sl], sem))
                for c in copies:
                    c.wait()
                dst = pl.ds((b * P0 + pbase) * SH, _HROWS)
                pltpu.sync_copy(rlo_v, glo_hbm.at[dst])
                pltpu.sync_copy(rhi_v, ghi_hbm.at[dst])
            return carry

        lax.fori_loop(0, _NCH, chunk_body, 0)

    return enc_gather


_enc_gather_halves = (_make_enc_gather(0), _make_enc_gather(BH))


@functools.partial(
    pl.kernel,
    mesh=_sc_mesh(),
    out_type=jax.ShapeDtypeStruct((BH * P0 // 8, 8 * F_OUT), jnp.float32),
    compiler_params=pltpu.CompilerParams(use_tc_tiling_on_sc=False),
    scratch_types=[
        pltpu.VMEM((_HROWS,), jnp.int32),
        pltpu.VMEM((_HROWS,), jnp.int32),
        pltpu.VMEM((_HROWS, F_OUT), jnp.float32),
        pltpu.VMEM((_HROWS, F_OUT), jnp.float32),
        pltpu.VMEM((_PCH // 8, 8 * F_OUT), jnp.float32),
        pltpu.VMEM((F_OUT,), jnp.float32),
        pltpu.SemaphoreType.DMA,
    ],
)
def _sc_dec_bag(ylo_hbm, yhi_hbm, splo_hbm, sphi_hbm, bdec_hbm, out_hbm,
                ilo_v, ihi_v, rlo_v, rhi_v, out_v, bias_v, sem):
    wid = _worker_id()
    pltpu.sync_copy(bdec_hbm, bias_v)
    lane8 = jnp.bitwise_and(lax.iota(jnp.int32, 16), 7)

    def chunk_body(pc, carry):
        pbase = wid * _PW + pc * _PCH
        pltpu.sync_copy(splo_hbm.at[pl.ds(pbase * SH, _HROWS)], ilo_v)
        pltpu.sync_copy(sphi_hbm.at[pl.ds(pbase * SH, _HROWS)], ihi_v)

        def mk_idx(r, rcarry):
            sl = pl.ds(r * 16, 16)
            ilo_v[sl] = ilo_v[sl] * SH + lane8
            ihi_v[sl] = ihi_v[sl] * SH + lane8
            return rcarry

        lax.fori_loop(0, _HROWS // 16, mk_idx, 0)
        bias = bias_v[...]
        for b in range(BH):
            if b > 0:
                def bump(r, rcarry):
                    sl = pl.ds(r * 16, 16)
                    ilo_v[sl] = ilo_v[sl] + (P0 * SH)
                    ihi_v[sl] = ihi_v[sl] + (P0 * SH)
                    return rcarry
                lax.fori_loop(0, _HROWS // 16, bump, 0)
            copies = []
            for j in range(_FIRES):
                sl = pl.ds(j * _STREAM, _STREAM)
                copies.append(pltpu.async_copy(
                    ylo_hbm.at[ilo_v.at[sl]], rlo_v.at[sl], sem))
                copies.append(pltpu.async_copy(
                    yhi_hbm.at[ihi_v.at[sl]], rhi_v.at[sl], sem))
            for c in copies:
                c.wait()

            def row_body(r, rcarry):
                acc = bias
                for j in range(SH):
                    acc = acc + rlo_v[r * SH + j]
                for j in range(SH):
                    acc = acc + rhi_v[r * SH + j]
                keep = jnp.where(pbase + r == P0 - 1,
                                 jnp.float32(0.0), jnp.float32(1.0))
                out_v[r // 8, pl.ds((r % 8) * F_OUT, F_OUT)] = acc * keep
                return rcarry

            lax.fori_loop(0, _PCH, row_body, 0)
            pltpu.sync_copy(
                out_v, out_hbm.at[pl.ds((b * P0 + pbase) // 8, _PCH // 8)])
        return carry

    lax.fori_loop(0, _NCH, chunk_body, 0)


_TKA = 2048
_KBA = P0 // _TKA


def _tc_a_body(glo_ref, ghi_ref, wlo_ref, whi_ref, b_ref, d_ref, o_ref,
               dc_ref):
    k = pl.program_id(0)
    b = pl.program_id(1)

    @pl.when(b == 0)
    def _():
        dc_ref[...] = d_ref[...].astype(jnp.bfloat16)

    h = lax.dot_general(glo_ref[...].astype(jnp.bfloat16),
                        wlo_ref[...].astype(jnp.bfloat16),
                        (((1,), (1,)), ((), ())),
                        preferred_element_type=jnp.float32)
    h = h + lax.dot_general(ghi_ref[...].astype(jnp.bfloat16),
                            whi_ref[...].astype(jnp.bfloat16),
                            (((1,), (1,)), ((), ())),
                            preferred_element_type=jnp.float32)
    h = h + b_ref[...]
    h = jnp.where(h > 0, h, jnp.exp(jnp.minimum(h, 0.0)) - 1.0)
    row = lax.broadcasted_iota(jnp.int32, (_TKA, 1), 0)
    h = jnp.where((k == _KBA - 1) & (row == _TKA - 1), 0.0, h)
    c = lax.dot_general(dc_ref[...], h.astype(jnp.bfloat16),
                        (((1,), (0,)), ((), ())),
                        preferred_element_type=jnp.float32)

    @pl.when(k == 0)
    def _():
        o_ref[b] = c

    @pl.when(k != 0)
    def _():
        o_ref[b] = o_ref[b] + c


def _tc_a(glo, ghi, w_lo, w_hi, b_enc2, d0m, nb):
    def rows(k, b):
        return (b * _KBA + k, 0)

    return pl.pallas_call(
        _tc_a_body,
        grid=(_KBA, nb),
        in_specs=[
            pl.BlockSpec((_TKA, SH * F_IN), rows),
            pl.BlockSpec((_TKA, SH * F_IN), rows),
            pl.BlockSpec((F_ENC, SH * F_IN), lambda k, b: (0, 0)),
            pl.BlockSpec((F_ENC, SH * F_IN), lambda k, b: (0, 0)),
            pl.BlockSpec((1, F_ENC), lambda k, b: (0, 0)),
            pl.BlockSpec((P1, _TKA), lambda k, b: (0, k)),
        ],
        out_specs=pl.BlockSpec((nb, P1, F_ENC), lambda k, b: (0, 0, 0)),
        out_shape=jax.ShapeDtypeStruct((nb, P1, F_ENC), jnp.float32),
        scratch_shapes=[pltpu.VMEM((P1, _TKA), jnp.bfloat16)],
    )(glo, ghi, w_lo, w_hi, b_enc2, d0m)


_CKB = 8192
_KBB = (P1 * F_ENC) // _CKB


def _tc_b1_body(ha_ref, hb_ref, w_ref, b_ref, o_ref):
    k = pl.program_id(0)
    h = jnp.concatenate([ha_ref[...], hb_ref[...]], axis=0)
    c = lax.dot_general(h, w_ref[...], (((1,), (1,)), ((), ())),
                        preferred_element_type=jnp.float32)

    @pl.when(k == 0)
    def _():
        o_ref[...] = c + b_ref[...]

    @pl.when(k != 0)
    def _():
        o_ref[...] = o_ref[...] + c


def _tc_b1(hda, hdb, w_fc_enc, b_fc_enc2):
    return pl.pallas_call(
        _tc_b1_body,
        grid=(_KBB,),
        in_specs=[
            pl.BlockSpec((BH, _CKB), lambda k: (0, k)),
            pl.BlockSpec((BH, _CKB), lambda k: (0, k)),
            pl.BlockSpec((LATENT, _CKB), lambda k: (0, k)),
            pl.BlockSpec((1, LATENT), lambda k: (0, 0)),
        ],
        out_specs=pl.BlockSpec((B, LATENT), lambda k: (0, 0)),
        out_shape=jax.ShapeDtypeStruct((B, LATENT), jnp.float32),
    )(hda, hdb, w_fc_enc, b_fc_enc2)


_CNB = 8192
_NBB = (P1 * F_DEC0) // _CNB


def _tc_b2_body(z_ref, w_ref, b_ref, o_ref):
    c = lax.dot_general(z_ref[...], w_ref[...], (((1,), (1,)), ((), ())),
                        preferred_element_type=jnp.float32)
    o_ref[...] = c + b_ref[...]


def _tc_b2(z, w_fc_dec, b_fc_dec2):
    return pl.pallas_call(
        _tc_b2_body,
        grid=(_NBB,),
        in_specs=[
            pl.BlockSpec((B, LATENT), lambda n: (0, 0)),
            pl.BlockSpec((_CNB, LATENT), lambda n: (n, 0)),
            pl.BlockSpec((1, _CNB), lambda n: (0, n)),
        ],
        out_specs=pl.BlockSpec((B, _CNB), lambda n: (0, n)),
        out_shape=jax.ShapeDtypeStruct((B, P1 * F_DEC0), jnp.float32),
    )(z, w_fc_dec, b_fc_dec2)


_TMC = 2048
_MBC = P0 // _TMC


def _tc_c_body(u_ref, d_ref, wlo_ref, whi_ref, olo_ref, ohi_ref, uc_ref):
    b = pl.program_id(1)

    @pl.when(b == 0)
    def _():
        uc_ref[...] = u_ref[...].astype(jnp.bfloat16)

    u = lax.dot_general(uc_ref[...], d_ref[0].astype(jnp.bfloat16),
                        (((1,), (0,)), ((), ())),
                        preferred_element_type=jnp.float32)
    ub = u.astype(jnp.bfloat16)
    olo_ref[...] = lax.dot_general(ub, wlo_ref[...].astype(jnp.bfloat16),
                                   (((1,), (0,)), ((), ())),
                                   preferred_element_type=jnp.float32)
    ohi_ref[...] = lax.dot_general(ub, whi_ref[...].astype(jnp.bfloat16),
                                   (((1,), (0,)), ((), ())),
                                   preferred_element_type=jnp.float32)


def _tc_c(u0, dd3, wd2lo, wd2hi):
    nb = dd3.shape[0]

    def rows(m, b):
        return (b * _MBC + m, 0)

    return pl.pallas_call(
        _tc_c_body,
        grid=(_MBC, nb),
        in_specs=[
            pl.BlockSpec((_TMC, P1), lambda m, b: (m, 0)),
            pl.BlockSpec((1, P1, F_DEC0), lambda m, b: (b, 0, 0)),
            pl.BlockSpec((F_DEC0, SH * F_OUT), lambda m, b: (0, 0)),
            pl.BlockSpec((F_DEC0, SH * F_OUT), lambda m, b: (0, 0)),
        ],
        out_specs=(pl.BlockSpec((_TMC, SH * F_OUT), rows),
                   pl.BlockSpec((_TMC, SH * F_OUT), rows)),
        out_shape=(jax.ShapeDtypeStruct((nb * P0, SH * F_OUT), jnp.float32),
                   jax.ShapeDtypeStruct((nb * P0, SH * F_OUT), jnp.float32)),
        scratch_shapes=[pltpu.VMEM((_TMC, P1), jnp.bfloat16)],
    )(u0, dd3, wd2lo, wd2hi)


def kernel(x, spirals0, W_enc, b_enc, D0, W_fc_enc, b_fc_enc, W_fc_dec,
           b_fc_dec, U0, W_dec, b_dec):
    sp = spirals0.astype(jnp.int32)
    sp_lo = sp[:, :SH].reshape(-1)
    sp_hi = sp[:, SH:].reshape(-1)

    wd2 = W_dec.reshape(F_OUT, S, F_DEC0).transpose(2, 1, 0).reshape(
        F_DEC0, S * F_OUT)
    wd2lo, wd2hi = wd2[:, :SH * F_OUT], wd2[:, SH * F_OUT:]
    w_lo, w_hi = W_enc[:, :SH * F_IN], W_enc[:, SH * F_IN:]
    b_enc2 = b_enc.reshape(1, F_ENC)

    xlin = x.reshape(B * P0, F_IN)

    gla, gha = _enc_gather_halves[0](xlin, sp_lo, sp_hi)
    glb, ghb = _enc_gather_halves[1](xlin, sp_lo, sp_hi)
    hda = _tc_a(gla.reshape(BH * P0, SH * F_IN),
                gha.reshape(BH * P0, SH * F_IN), w_lo, w_hi, b_enc2, D0, BH)
    hdb = _tc_a(glb.reshape(BH * P0, SH * F_IN),
                ghb.reshape(BH * P0, SH * F_IN), w_lo, w_hi, b_enc2, D0, BH)

    z = _tc_b1(hda.reshape(BH, P1 * F_ENC), hdb.reshape(BH, P1 * F_ENC),
               W_fc_enc, b_fc_enc.reshape(1, LATENT))
    dd = _tc_b2(z, W_fc_dec, b_fc_dec.reshape(1, P1 * F_DEC0))
    dd3 = dd.reshape(B, P1, F_DEC0)

    yla, yha = _tc_c(U0, dd3[:BH], wd2lo, wd2hi)
    ylb, yhb = _tc_c(U0, dd3[BH:], wd2lo, wd2hi)
    outa = _sc_dec_bag(yla.reshape(BH * P0 * SH, F_OUT),
                       yha.reshape(BH * P0 * SH, F_OUT), sp_lo, sp_hi, b_dec)
    outb = _sc_dec_bag(ylb.reshape(BH * P0 * SH, F_OUT),
                       yhb.reshape(BH * P0 * SH, F_OUT), sp_lo, sp_hi, b_dec)
    out = jnp.concatenate([outa, outb], axis=0)
    return out.reshape(B, P0, F_OUT)


# speedup vs baseline: 1.4561x; 1.0558x over previous
import functools

import jax
import jax.numpy as jnp
from jax import lax
from jax.experimental import pallas as pl
from jax.experimental.pallas import tpu as pltpu, tpu_sc as plsc

B = 8
BH = 4
P0 = 16384
P1 = 1024
S = 16
SH = 8
F_IN = 16
F_ENC = 128
LATENT = 256
F_DEC0 = 128
F_OUT = 16

NW = 32
_STREAM = 128
_PW = P0 // NW
_PCH = 128
_NCH = _PW // _PCH
_HROWS = _PCH * SH
_FIRES = _HROWS // _STREAM


def _sc_mesh():
    return plsc.VectorSubcoreMesh(core_axis_name="c", subcore_axis_name="s")


def _worker_id():
    return lax.axis_index("s") * 2 + lax.axis_index("c")


def _make_enc_gather(b0):
    @functools.partial(
        pl.kernel,
        mesh=_sc_mesh(),
        out_type=(jax.ShapeDtypeStruct((BH * P0 * SH, F_IN), jnp.float32),
                  jax.ShapeDtypeStruct((BH * P0 * SH, F_IN), jnp.float32)),
        compiler_params=pltpu.CompilerParams(use_tc_tiling_on_sc=False),
        scratch_types=[
            pltpu.VMEM((_HROWS,), jnp.int32),
            pltpu.VMEM((_HROWS,), jnp.int32),
            pltpu.VMEM((_HROWS, F_IN), jnp.float32),
            pltpu.VMEM((_HROWS, F_IN), jnp.float32),
            pltpu.SemaphoreType.DMA,
        ],
    )
    def enc_gather(x_hbm, splo_hbm, sphi_hbm, glo_hbm, ghi_hbm,
                   splo_v, sphi_v, rlo_v, rhi_v, sem):
        wid = _worker_id()
        nvec = _HROWS // 16

        def chunk_body(pc, carry):
            pbase = wid * _PW + pc * _PCH
            pltpu.sync_copy(splo_hbm.at[pl.ds(pbase * SH, _HROWS)], splo_v)
            pltpu.sync_copy(sphi_hbm.at[pl.ds(pbase * SH, _HROWS)], sphi_v)
            for b in range(BH):
                off = (b0 if b == 0 else 0) * P0 + (P0 if b > 0 else 0)
                if off:
                    def bump(r, rcarry):
                        sl = pl.ds(r * 16, 16)
                        splo_v[sl] = splo_v[sl] + off
                        sphi_v[sl] = sphi_v[sl] + off
                        return rcarry
                    lax.fori_loop(0, nvec, bump, 0)
                copies = []
                for j in range(_FIRES):
                    sl = pl.ds(j * _STREAM, _STREAM)
                    copies.append(pltpu.async_copy(
                        x_hbm.at[splo_v.at[sl]], rlo_v.at[sl], sem))
                    copies.append(pltpu.async_copy(
                        x_hbm.at[sphi_v.at[sl]], rhi_v.at[sl], sem))
                for c in copies:
                    c.wait()
                dst = pl.ds((b * P0 + pbase) * SH, _HROWS)
                pltpu.sync_copy(rlo_v, glo_hbm.at[dst])
                pltpu.sync_copy(rhi_v, ghi_hbm.at[dst])
            return carry

        lax.fori_loop(0, _NCH, chunk_body, 0)

    return enc_gather


_enc_gather_halves = (_make_enc_gather(0), _make_enc_gather(BH))


@functools.partial(
    pl.kernel,
    mesh=_sc_mesh(),
    out_type=jax.ShapeDtypeStruct((BH * P0 // 8, 8 * F_OUT), jnp.float32),
    compiler_params=pltpu.CompilerParams(use_tc_tiling_on_sc=False),
    scratch_types=[
        pltpu.VMEM((_HROWS,), jnp.int32),
        pltpu.VMEM((_HROWS,), jnp.int32),
        pltpu.VMEM((_HROWS,), jnp.int32),
        pltpu.VMEM((_HROWS,), jnp.int32),
        pltpu.VMEM((_HROWS, F_OUT), jnp.float32),
        pltpu.VMEM((_HROWS, F_OUT), jnp.float32),
        pltpu.VMEM((_HROWS, F_OUT), jnp.float32),
        pltpu.VMEM((_HROWS, F_OUT), jnp.float32),
        pltpu.VMEM((_PCH // 8, 8 * F_OUT), jnp.float32),
        pltpu.VMEM((F_OUT,), jnp.float32),
        pltpu.SemaphoreType.DMA,
        pltpu.SemaphoreType.DMA,
    ],
)
def _sc_dec_bag(ylo_hbm, yhi_hbm, splo_hbm, sphi_hbm, bdec_hbm, out_hbm,
                ilo0_v, ihi0_v, ilo1_v, ihi1_v,
                rlo0_v, rhi0_v, rlo1_v, rhi1_v, out_v, bias_v, sem0, sem1):
    wid = _worker_id()
    pltpu.sync_copy(bdec_hbm, bias_v)
    lane8 = jnp.bitwise_and(lax.iota(jnp.int32, 16), 7)
    ilo = (ilo0_v, ilo1_v)
    ihi = (ihi0_v, ihi1_v)
    rlo = (rlo0_v, rlo1_v)
    rhi = (rhi0_v, rhi1_v)
    sems = (sem0, sem1)

    def fire(par):
        copies = []
        for j in range(_FIRES):
            sl = pl.ds(j * _STREAM, _STREAM)
            copies.append(pltpu.async_copy(
                ylo_hbm.at[ilo[par].at[sl]], rlo[par].at[sl], sems[par]))
            copies.append(pltpu.async_copy(
                yhi_hbm.at[ihi[par].at[sl]], rhi[par].at[sl], sems[par]))
        return copies

    def chunk_body(pc, carry):
        pbase = wid * _PW + pc * _PCH
        pltpu.sync_copy(splo_hbm.at[pl.ds(pbase * SH, _HROWS)], ilo0_v)
        pltpu.sync_copy(sphi_hbm.at[pl.ds(pbase * SH, _HROWS)], ihi0_v)

        def mk_idx(r, rcarry):
            sl = pl.ds(r * 16, 16)
            ilo0_v[sl] = ilo0_v[sl] * SH + lane8
            ihi0_v[sl] = ihi0_v[sl] * SH + lane8
            return rcarry

        lax.fori_loop(0, _HROWS // 16, mk_idx, 0)
        bias = bias_v[...]
        inflight = fire(0)
        for b in range(BH):
            par = b % 2
            nxt = 1 - par
            if b < BH - 1:
                def bump(r, rcarry):
                    sl = pl.ds(r * 16, 16)
                    ilo[nxt][sl] = ilo[par][sl] + (P0 * SH)
                    ihi[nxt][sl] = ihi[par][sl] + (P0 * SH)
                    return rcarry
                lax.fori_loop(0, _HROWS // 16, bump, 0)
                nxt_copies = fire(nxt)
            else:
                nxt_copies = None
            for c in inflight:
                c.wait()
            inflight = nxt_copies

            def row_body(r, rcarry):
                acc = bias
                for j in range(SH):
                    acc = acc + rlo[par][r * SH + j]
                for j in range(SH):
                    acc = acc + rhi[par][r * SH + j]
                keep = jnp.where(pbase + r == P0 - 1,
                                 jnp.float32(0.0), jnp.float32(1.0))
                out_v[r // 8, pl.ds((r % 8) * F_OUT, F_OUT)] = acc * keep
                return rcarry

            lax.fori_loop(0, _PCH, row_body, 0)
            pltpu.sync_copy(
                out_v, out_hbm.at[pl.ds((b * P0 + pbase) // 8, _PCH // 8)])
        return carry

    lax.fori_loop(0, _NCH, chunk_body, 0)


_TKA = 2048
_KBA = P0 // _TKA


def _tc_a_body(glo_ref, ghi_ref, wlo_ref, whi_ref, b_ref, d_ref, o_ref,
               dc_ref):
    k = pl.program_id(0)
    b = pl.program_id(1)

    @pl.when(b == 0)
    def _():
        dc_ref[...] = d_ref[...].astype(jnp.bfloat16)

    h = lax.dot_general(glo_ref[...].astype(jnp.bfloat16),
                        wlo_ref[...].astype(jnp.bfloat16),
                        (((1,), (1,)), ((), ())),
                        preferred_element_type=jnp.float32)
    h = h + lax.dot_general(ghi_ref[...].astype(jnp.bfloat16),
                            whi_ref[...].astype(jnp.bfloat16),
                            (((1,), (1,)), ((), ())),
                            preferred_element_type=jnp.float32)
    h = h + b_ref[...]
    h = jnp.where(h > 0, h, jnp.exp(jnp.minimum(h, 0.0)) - 1.0)
    row = lax.broadcasted_iota(jnp.int32, (_TKA, 1), 0)
    h = jnp.where((k == _KBA - 1) & (row == _TKA - 1), 0.0, h)
    c = lax.dot_general(dc_ref[...], h.astype(jnp.bfloat16),
                        (((1,), (0,)), ((), ())),
                        preferred_element_type=jnp.float32)

    @pl.when(k == 0)
    def _():
        o_ref[b] = c

    @pl.when(k != 0)
    def _():
        o_ref[b] = o_ref[b] + c


def _tc_a(glo, ghi, w_lo, w_hi, b_enc2, d0m, nb):
    def rows(k, b):
        return (b * _KBA + k, 0)

    return pl.pallas_call(
        _tc_a_body,
        grid=(_KBA, nb),
        in_specs=[
            pl.BlockSpec((_TKA, SH * F_IN), rows),
            pl.BlockSpec((_TKA, SH * F_IN), rows),
            pl.BlockSpec((F_ENC, SH * F_IN), lambda k, b: (0, 0)),
            pl.BlockSpec((F_ENC, SH * F_IN), lambda k, b: (0, 0)),
            pl.BlockSpec((1, F_ENC), lambda k, b: (0, 0)),
            pl.BlockSpec((P1, _TKA), lambda k, b: (0, k)),
        ],
        out_specs=pl.BlockSpec((nb, P1, F_ENC), lambda k, b: (0, 0, 0)),
        out_shape=jax.ShapeDtypeStruct((nb, P1, F_ENC), jnp.float32),
        scratch_shapes=[pltpu.VMEM((P1, _TKA), jnp.bfloat16)],
    )(glo, ghi, w_lo, w_hi, b_enc2, d0m)


_CKB = 8192
_KBB = (P1 * F_ENC) // _CKB


def _tc_b1_body(ha_ref, hb_ref, w_ref, b_ref, o_ref):
    k = pl.program_id(0)
    h = jnp.concatenate([ha_ref[...], hb_ref[...]], axis=0)
    c = lax.dot_general(h, w_ref[...], (((1,), (1,)), ((), ())),
                        preferred_element_type=jnp.float32)

    @pl.when(k == 0)
    def _():
        o_ref[...] = c + b_ref[...]

    @pl.when(k != 0)
    def _():
        o_ref[...] = o_ref[...] + c


def _tc_b1(hda, hdb, w_fc_enc, b_fc_enc2):
    return pl.pallas_call(
        _tc_b1_body,
        grid=(_KBB,),
        in_specs=[
            pl.BlockSpec((BH, _CKB), lambda k: (0, k)),
            pl.BlockSpec((BH, _CKB), lambda k: (0, k)),
            pl.BlockSpec((LATENT, _CKB), lambda k: (0, k)),
            pl.BlockSpec((1, LATENT), lambda k: (0, 0)),
        ],
        out_specs=pl.BlockSpec((B, LATENT), lambda k: (0, 0)),
        out_shape=jax.ShapeDtypeStruct((B, LATENT), jnp.float32),
    )(hda, hdb, w_fc_enc, b_fc_enc2)


_CNB = 8192
_NBB = (P1 * F_DEC0) // _CNB


def _tc_b2_body(z_ref, w_ref, b_ref, o_ref):
    c = lax.dot_general(z_ref[...], w_ref[...], (((1,), (1,)), ((), ())),
                        preferred_element_type=jnp.float32)
    o_ref[...] = c + b_ref[...]


def _tc_b2(z, w_fc_dec, b_fc_dec2):
    return pl.pallas_call(
        _tc_b2_body,
        grid=(_NBB,),
        in_specs=[
            pl.BlockSpec((B, LATENT), lambda n: (0, 0)),
            pl.BlockSpec((_CNB, LATENT), lambda n: (n, 0)),
            pl.BlockSpec((1, _CNB), lambda n: (0, n)),
        ],
        out_specs=pl.BlockSpec((B, _CNB), lambda n: (0, n)),
        out_shape=jax.ShapeDtypeStruct((B, P1 * F_DEC0), jnp.float32),
    )(z, w_fc_dec, b_fc_dec2)


_TMC = 2048
_MBC = P0 // _TMC


def _tc_c_body(u_ref, d_ref, wlo_ref, whi_ref, olo_ref, ohi_ref, uc_ref):
    b = pl.program_id(1)

    @pl.when(b == 0)
    def _():
        uc_ref[...] = u_ref[...].astype(jnp.bfloat16)

    u = lax.dot_general(uc_ref[...], d_ref[0].astype(jnp.bfloat16),
                        (((1,), (0,)), ((), ())),
                        preferred_element_type=jnp.float32)
    ub = u.astype(jnp.bfloat16)
    olo_ref[...] = lax.dot_general(ub, wlo_ref[...].astype(jnp.bfloat16),
                                   (((1,), (0,)), ((), ())),
                                   preferred_element_type=jnp.float32)
    ohi_ref[...] = lax.dot_general(ub, whi_ref[...].astype(jnp.bfloat16),
                                   (((1,), (0,)), ((), ())),
                                   preferred_element_type=jnp.float32)


def _tc_c(u0, dd3, wd2lo, wd2hi):
    nb = dd3.shape[0]

    def rows(m, b):
        return (b * _MBC + m, 0)

    return pl.pallas_call(
        _tc_c_body,
        grid=(_MBC, nb),
        in_specs=[
            pl.BlockSpec((_TMC, P1), lambda m, b: (m, 0)),
            pl.BlockSpec((1, P1, F_DEC0), lambda m, b: (b, 0, 0)),
            pl.BlockSpec((F_DEC0, SH * F_OUT), lambda m, b: (0, 0)),
            pl.BlockSpec((F_DEC0, SH * F_OUT), lambda m, b: (0, 0)),
        ],
        out_specs=(pl.BlockSpec((_TMC, SH * F_OUT), rows),
                   pl.BlockSpec((_TMC, SH * F_OUT), rows)),
        out_shape=(jax.ShapeDtypeStruct((nb * P0, SH * F_OUT), jnp.float32),
                   jax.ShapeDtypeStruct((nb * P0, SH * F_OUT), jnp.float32)),
        scratch_shapes=[pltpu.VMEM((_TMC, P1), jnp.bfloat16)],
    )(u0, dd3, wd2lo, wd2hi)


def kernel(x, spirals0, W_enc, b_enc, D0, W_fc_enc, b_fc_enc, W_fc_dec,
           b_fc_dec, U0, W_dec, b_dec):
    sp = spirals0.astype(jnp.int32)
    sp_lo = sp[:, :SH].reshape(-1)
    sp_hi = sp[:, SH:].reshape(-1)

    wd2 = W_dec.reshape(F_OUT, S, F_DEC0).transpose(2, 1, 0).reshape(
        F_DEC0, S * F_OUT)
    wd2lo, wd2hi = wd2[:, :SH * F_OUT], wd2[:, SH * F_OUT:]
    w_lo, w_hi = W_enc[:, :SH * F_IN], W_enc[:, SH * F_IN:]
    b_enc2 = b_enc.reshape(1, F_ENC)

    xlin = x.reshape(B * P0, F_IN)

    gla, gha = _enc_gather_halves[0](xlin, sp_lo, sp_hi)
    glb, ghb = _enc_gather_halves[1](xlin, sp_lo, sp_hi)
    hda = _tc_a(gla.reshape(BH * P0, SH * F_IN),
                gha.reshape(BH * P0, SH * F_IN), w_lo, w_hi, b_enc2, D0, BH)
    hdb = _tc_a(glb.reshape(BH * P0, SH * F_IN),
                ghb.reshape(BH * P0, SH * F_IN), w_lo, w_hi, b_enc2, D0, BH)

    z = _tc_b1(hda.reshape(BH, P1 * F_ENC), hdb.reshape(BH, P1 * F_ENC),
               W_fc_enc, b_fc_enc.reshape(1, LATENT))
    dd = _tc_b2(z, W_fc_dec, b_fc_dec.reshape(1, P1 * F_DEC0))
    dd3 = dd.reshape(B, P1, F_DEC0)

    yla, yha = _tc_c(U0, dd3[:BH], wd2lo, wd2hi)
    ylb, yhb = _tc_c(U0, dd3[BH:], wd2lo, wd2hi)
    outa = _sc_dec_bag(yla.reshape(BH * P0 * SH, F_OUT),
                       yha.reshape(BH * P0 * SH, F_OUT), sp_lo, sp_hi, b_dec)
    outb = _sc_dec_bag(ylb.reshape(BH * P0 * SH, F_OUT),
                       yhb.reshape(BH * P0 * SH, F_OUT), sp_lo, sp_hi, b_dec)
    out = jnp.concatenate([outa, outb], axis=0)
    return out.reshape(B, P0, F_OUT)


# speedup vs baseline: 1.4825x; 1.0181x over previous
import functools

import jax
import jax.numpy as jnp
from jax import lax
from jax.experimental import pallas as pl
from jax.experimental.pallas import tpu as pltpu, tpu_sc as plsc

B = 8
BH = 4
P0 = 16384
P1 = 1024
S = 16
SH = 8
F_IN = 16
F_ENC = 128
LATENT = 256
F_DEC0 = 128
F_OUT = 16

NW = 32
_STREAM = 128
_PW = P0 // NW
_PCH = 128
_NCH = _PW // _PCH
_HROWS = _PCH * SH
_FIRES = _HROWS // _STREAM


def _sc_mesh():
    return plsc.VectorSubcoreMesh(core_axis_name="c", subcore_axis_name="s")


def _worker_id():
    return lax.axis_index("s") * 2 + lax.axis_index("c")


def _make_enc_gather(b0):
    @functools.partial(
        pl.kernel,
        mesh=_sc_mesh(),
        out_type=(jax.ShapeDtypeStruct((BH * P0 * SH, F_IN), jnp.float32),
                  jax.ShapeDtypeStruct((BH * P0 * SH, F_IN), jnp.float32)),
        compiler_params=pltpu.CompilerParams(use_tc_tiling_on_sc=False),
        scratch_types=[
            pltpu.VMEM((_HROWS,), jnp.int32),
            pltpu.VMEM((_HROWS,), jnp.int32),
            pltpu.VMEM((_HROWS,), jnp.int32),
            pltpu.VMEM((_HROWS,), jnp.int32),
            pltpu.VMEM((_HROWS, F_IN), jnp.float32),
            pltpu.VMEM((_HROWS, F_IN), jnp.float32),
            pltpu.VMEM((_HROWS, F_IN), jnp.float32),
            pltpu.VMEM((_HROWS, F_IN), jnp.float32),
            pltpu.SemaphoreType.DMA,
            pltpu.SemaphoreType.DMA,
        ],
    )
    def enc_gather(x_hbm, splo_hbm, sphi_hbm, glo_hbm, ghi_hbm,
                   slo0_v, shi0_v, slo1_v, shi1_v,
                   rlo0_v, rhi0_v, rlo1_v, rhi1_v, sem0, sem1):
        wid = _worker_id()
        nvec = _HROWS // 16
        slo = (slo0_v, slo1_v)
        shi = (shi0_v, shi1_v)
        rlo = (rlo0_v, rlo1_v)
        rhi = (rhi0_v, rhi1_v)
        sems = (sem0, sem1)

        def fire(par):
            copies = []
            for j in range(_FIRES):
                sl = pl.ds(j * _STREAM, _STREAM)
                copies.append(pltpu.async_copy(
                    x_hbm.at[slo[par].at[sl]], rlo[par].at[sl], sems[par]))
                copies.append(pltpu.async_copy(
                    x_hbm.at[shi[par].at[sl]], rhi[par].at[sl], sems[par]))
            return copies

        def chunk_body(pc, carry):
            pbase = wid * _PW + pc * _PCH
            pltpu.sync_copy(splo_hbm.at[pl.ds(pbase * SH, _HROWS)], slo0_v)
            pltpu.sync_copy(sphi_hbm.at[pl.ds(pbase * SH, _HROWS)], shi0_v)
            if b0:
                def base_bump(r, rcarry):
                    sl = pl.ds(r * 16, 16)
                    slo0_v[sl] = slo0_v[sl] + b0 * P0
                    shi0_v[sl] = shi0_v[sl] + b0 * P0
                    return rcarry
                lax.fori_loop(0, nvec, base_bump, 0)
            inflight = fire(0)
            for b in range(BH):
                par = b % 2
                nxt = 1 - par
                if b < BH - 1:
                    def bump(r, rcarry):
                        sl = pl.ds(r * 16, 16)
                        slo[nxt][sl] = slo[par][sl] + P0
                        shi[nxt][sl] = shi[par][sl] + P0
                        return rcarry
                    lax.fori_loop(0, nvec, bump, 0)
                    nxt_copies = fire(nxt)
                else:
                    nxt_copies = None
                for c in inflight:
                    c.wait()
                inflight = nxt_copies
                dst = pl.ds((b * P0 + pbase) * SH, _HROWS)
                pltpu.sync_copy(rlo[par], glo_hbm.at[dst])
                pltpu.sync_copy(rhi[par], ghi_hbm.at[dst])
            return carry

        lax.fori_loop(0, _NCH, chunk_body, 0)

    return enc_gather


_enc_gather_halves = (_make_enc_gather(0), _make_enc_gather(BH))


@functools.partial(
    pl.kernel,
    mesh=_sc_mesh(),
    out_type=jax.ShapeDtypeStruct((BH * P0 // 8, 8 * F_OUT), jnp.float32),
    compiler_params=pltpu.CompilerParams(use_tc_tiling_on_sc=False),
    scratch_types=[
        pltpu.VMEM((_HROWS,), jnp.int32),
        pltpu.VMEM((_HROWS,), jnp.int32),
        pltpu.VMEM((_HROWS,), jnp.int32),
        pltpu.VMEM((_HROWS,), jnp.int32),
        pltpu.VMEM((_HROWS, F_OUT), jnp.float32),
        pltpu.VMEM((_HROWS, F_OUT), jnp.float32),
        pltpu.VMEM((_HROWS, F_OUT), jnp.float32),
        pltpu.VMEM((_HROWS, F_OUT), jnp.float32),
        pltpu.VMEM((_PCH // 8, 8 * F_OUT), jnp.float32),
        pltpu.VMEM((F_OUT,), jnp.float32),
        pltpu.SemaphoreType.DMA,
        pltpu.SemaphoreType.DMA,
    ],
)
def _sc_dec_bag(ylo_hbm, yhi_hbm, splo_hbm, sphi_hbm, bdec_hbm, out_hbm,
                ilo0_v, ihi0_v, ilo1_v, ihi1_v,
                rlo0_v, rhi0_v, rlo1_v, rhi1_v, out_v, bias_v, sem0, sem1):
    wid = _worker_id()
    pltpu.sync_copy(bdec_hbm, bias_v)
    lane8 = jnp.bitwise_and(lax.iota(jnp.int32, 16), 7)
    ilo = (ilo0_v, ilo1_v)
    ihi = (ihi0_v, ihi1_v)
    rlo = (rlo0_v, rlo1_v)
    rhi = (rhi0_v, rhi1_v)
    sems = (sem0, sem1)

    def fire(par):
        copies = []
        for j in range(_FIRES):
            sl = pl.ds(j * _STREAM, _STREAM)
            copies.append(pltpu.async_copy(
                ylo_hbm.at[ilo[par].at[sl]], rlo[par].at[sl], sems[par]))
            copies.append(pltpu.async_copy(
                yhi_hbm.at[ihi[par].at[sl]], rhi[par].at[sl], sems[par]))
        return copies

    def chunk_body(pc, carry):
        pbase = wid * _PW + pc * _PCH
        pltpu.sync_copy(splo_hbm.at[pl.ds(pbase * SH, _HROWS)], ilo0_v)
        pltpu.sync_copy(sphi_hbm.at[pl.ds(pbase * SH, _HROWS)], ihi0_v)

        def mk_idx(r, rcarry):
            sl = pl.ds(r * 16, 16)
            ilo0_v[sl] = ilo0_v[sl] * SH + lane8
            ihi0_v[sl] = ihi0_v[sl] * SH + lane8
            return rcarry

        lax.fori_loop(0, _HROWS // 16, mk_idx, 0)
        bias = bias_v[...]
        inflight = fire(0)
        for b in range(BH):
            par = b % 2
            nxt = 1 - par
            if b < BH - 1:
                def bump(r, rcarry):
                    sl = pl.ds(r * 16, 16)
                    ilo[nxt][sl] = ilo[par][sl] + (P0 * SH)
                    ihi[nxt][sl] = ihi[par][sl] + (P0 * SH)
                    return rcarry
                lax.fori_loop(0, _HROWS // 16, bump, 0)
                nxt_copies = fire(nxt)
            else:
                nxt_copies = None
            for c in inflight:
                c.wait()
            inflight = nxt_copies

            def row_body(r, rcarry):
                acc = bias
                for j in range(SH):
                    acc = acc + rlo[par][r * SH + j]
                for j in range(SH):
                    acc = acc + rhi[par][r * SH + j]
                keep = jnp.where(pbase + r == P0 - 1,
                                 jnp.float32(0.0), jnp.float32(1.0))
                out_v[r // 8, pl.ds((r % 8) * F_OUT, F_OUT)] = acc * keep
                return rcarry

            lax.fori_loop(0, _PCH, row_body, 0)
            pltpu.sync_copy(
                out_v, out_hbm.at[pl.ds((b * P0 + pbase) // 8, _PCH // 8)])
        return carry

    lax.fori_loop(0, _NCH, chunk_body, 0)


_TKA = 2048
_KBA = P0 // _TKA


def _tc_a_body(glo_ref, ghi_ref, wlo_ref, whi_ref, b_ref, d_ref, o_ref,
               dc_ref):
    k = pl.program_id(0)
    b = pl.program_id(1)

    @pl.when(b == 0)
    def _():
        dc_ref[...] = d_ref[...].astype(jnp.bfloat16)

    h = lax.dot_general(glo_ref[...].astype(jnp.bfloat16),
                        wlo_ref[...].astype(jnp.bfloat16),
                        (((1,), (1,)), ((), ())),
                        preferred_element_type=jnp.float32)
    h = h + lax.dot_general(ghi_ref[...].astype(jnp.bfloat16),
                            whi_ref[...].astype(jnp.bfloat16),
                            (((1,), (1,)), ((), ())),
                            preferred_element_type=jnp.float32)
    h = h + b_ref[...]
    h = jnp.where(h > 0, h, jnp.exp(jnp.minimum(h, 0.0)) - 1.0)
    row = lax.broadcasted_iota(jnp.int32, (_TKA, 1), 0)
    h = jnp.where((k == _KBA - 1) & (row == _TKA - 1), 0.0, h)
    c = lax.dot_general(dc_ref[...], h.astype(jnp.bfloat16),
                        (((1,), (0,)), ((), ())),
                        preferred_element_type=jnp.float32)

    @pl.when(k == 0)
    def _():
        o_ref[b] = c

    @pl.when(k != 0)
    def _():
        o_ref[b] = o_ref[b] + c


def _tc_a(glo, ghi, w_lo, w_hi, b_enc2, d0m, nb):
    def rows(k, b):
        return (b * _KBA + k, 0)

    return pl.pallas_call(
        _tc_a_body,
        grid=(_KBA, nb),
        in_specs=[
            pl.BlockSpec((_TKA, SH * F_IN), rows),
            pl.BlockSpec((_TKA, SH * F_IN), rows),
            pl.BlockSpec((F_ENC, SH * F_IN), lambda k, b: (0, 0)),
            pl.BlockSpec((F_ENC, SH * F_IN), lambda k, b: (0, 0)),
            pl.BlockSpec((1, F_ENC), lambda k, b: (0, 0)),
            pl.BlockSpec((P1, _TKA), lambda k, b: (0, k)),
        ],
        out_specs=pl.BlockSpec((nb, P1, F_ENC), lambda k, b: (0, 0, 0)),
        out_shape=jax.ShapeDtypeStruct((nb, P1, F_ENC), jnp.float32),
        scratch_shapes=[pltpu.VMEM((P1, _TKA), jnp.bfloat16)],
    )(glo, ghi, w_lo, w_hi, b_enc2, d0m)


_CKB = 8192
_KBB = (P1 * F_ENC) // _CKB


def _tc_b1_body(ha_ref, hb_ref, w_ref, b_ref, o_ref):
    k = pl.program_id(0)
    h = jnp.concatenate([ha_ref[...], hb_ref[...]], axis=0)
    c = lax.dot_general(h, w_ref[...], (((1,), (1,)), ((), ())),
                        preferred_element_type=jnp.float32)

    @pl.when(k == 0)
    def _():
        o_ref[...] = c + b_ref[...]

    @pl.when(k != 0)
    def _():
        o_ref[...] = o_ref[...] + c


def _tc_b1(hda, hdb, w_fc_enc, b_fc_enc2):
    return pl.pallas_call(
        _tc_b1_body,
        grid=(_KBB,),
        in_specs=[
            pl.BlockSpec((BH, _CKB), lambda k: (0, k)),
            pl.BlockSpec((BH, _CKB), lambda k: (0, k)),
            pl.BlockSpec((LATENT, _CKB), lambda k: (0, k)),
            pl.BlockSpec((1, LATENT), lambda k: (0, 0)),
        ],
        out_specs=pl.BlockSpec((B, LATENT), lambda k: (0, 0)),
        out_shape=jax.ShapeDtypeStruct((B, LATENT), jnp.float32),
    )(hda, hdb, w_fc_enc, b_fc_enc2)


_CNB = 8192
_NBB = (P1 * F_DEC0) // _CNB


def _tc_b2_body(z_ref, w_ref, b_ref, o_ref):
    c = lax.dot_general(z_ref[...], w_ref[...], (((1,), (1,)), ((), ())),
                        preferred_element_type=jnp.float32)
    o_ref[...] = c + b_ref[...]


def _tc_b2(z, w_fc_dec, b_fc_dec2):
    return pl.pallas_call(
        _tc_b2_body,
        grid=(_NBB,),
        in_specs=[
            pl.BlockSpec((B, LATENT), lambda n: (0, 0)),
            pl.BlockSpec((_CNB, LATENT), lambda n: (n, 0)),
            pl.BlockSpec((1, _CNB), lambda n: (0, n)),
        ],
        out_specs=pl.BlockSpec((B, _CNB), lambda n: (0, n)),
        out_shape=jax.ShapeDtypeStruct((B, P1 * F_DEC0), jnp.float32),
    )(z, w_fc_dec, b_fc_dec2)


_TMC = 2048
_MBC = P0 // _TMC


def _tc_c_body(u_ref, d_ref, wlo_ref, whi_ref, olo_ref, ohi_ref, uc_ref):
    b = pl.program_id(1)

    @pl.when(b == 0)
    def _():
        uc_ref[...] = u_ref[...].astype(jnp.bfloat16)

    u = lax.dot_general(uc_ref[...], d_ref[0].astype(jnp.bfloat16),
                        (((1,), (0,)), ((), ())),
                        preferred_element_type=jnp.float32)
    ub = u.astype(jnp.bfloat16)
    olo_ref[...] = lax.dot_general(ub, wlo_ref[...].astype(jnp.bfloat16),
                                   (((1,), (0,)), ((), ())),
                                   preferred_element_type=jnp.float32)
    ohi_ref[...] = lax.dot_general(ub, whi_ref[...].astype(jnp.bfloat16),
                                   (((1,), (0,)), ((), ())),
                                   preferred_element_type=jnp.float32)


def _tc_c(u0, dd3, wd2lo, wd2hi):
    nb = dd3.shape[0]

    def rows(m, b):
        return (b * _MBC + m, 0)

    return pl.pallas_call(
        _tc_c_body,
        grid=(_MBC, nb),
        in_specs=[
            pl.BlockSpec((_TMC, P1), lambda m, b: (m, 0)),
            pl.BlockSpec((1, P1, F_DEC0), lambda m, b: (b, 0, 0)),
            pl.BlockSpec((F_DEC0, SH * F_OUT), lambda m, b: (0, 0)),
            pl.BlockSpec((F_DEC0, SH * F_OUT), lambda m, b: (0, 0)),
        ],
        out_specs=(pl.BlockSpec((_TMC, SH * F_OUT), rows),
                   pl.BlockSpec((_TMC, SH * F_OUT), rows)),
        out_shape=(jax.ShapeDtypeStruct((nb * P0, SH * F_OUT), jnp.float32),
                   jax.ShapeDtypeStruct((nb * P0, SH * F_OUT), jnp.float32)),
        scratch_shapes=[pltpu.VMEM((_TMC, P1), jnp.bfloat16)],
    )(u0, dd3, wd2lo, wd2hi)


def kernel(x, spirals0, W_enc, b_enc, D0, W_fc_enc, b_fc_enc, W_fc_dec,
           b_fc_dec, U0, W_dec, b_dec):
    sp = spirals0.astype(jnp.int32)
    sp_lo = sp[:, :SH].reshape(-1)
    sp_hi = sp[:, SH:].reshape(-1)

    wd2 = W_dec.reshape(F_OUT, S, F_DEC0).transpose(2, 1, 0).reshape(
        F_DEC0, S * F_OUT)
    wd2lo, wd2hi = wd2[:, :SH * F_OUT], wd2[:, SH * F_OUT:]
    w_lo, w_hi = W_enc[:, :SH * F_IN], W_enc[:, SH * F_IN:]
    b_enc2 = b_enc.reshape(1, F_ENC)

    xlin = x.reshape(B * P0, F_IN)

    gla, gha = _enc_gather_halves[0](xlin, sp_lo, sp_hi)
    glb, ghb = _enc_gather_halves[1](xlin, sp_lo, sp_hi)
    hda = _tc_a(gla.reshape(BH * P0, SH * F_IN),
                gha.reshape(BH * P0, SH * F_IN), w_lo, w_hi, b_enc2, D0, BH)
    hdb = _tc_a(glb.reshape(BH * P0, SH * F_IN),
                ghb.reshape(BH * P0, SH * F_IN), w_lo, w_hi, b_enc2, D0, BH)

    z = _tc_b1(hda.reshape(BH, P1 * F_ENC), hdb.reshape(BH, P1 * F_ENC),
               W_fc_enc, b_fc_enc.reshape(1, LATENT))
    dd = _tc_b2(z, W_fc_dec, b_fc_dec.reshape(1, P1 * F_DEC0))
    dd3 = dd.reshape(B, P1, F_DEC0)

    yla, yha = _tc_c(U0, dd3[:BH], wd2lo, wd2hi)
    ylb, yhb = _tc_c(U0, dd3[BH:], wd2lo, wd2hi)
    outa = _sc_dec_bag(yla.reshape(BH * P0 * SH, F_OUT),
                       yha.reshape(BH * P0 * SH, F_OUT), sp_lo, sp_hi, b_dec)
    outb = _sc_dec_bag(ylb.reshape(BH * P0 * SH, F_OUT),
                       yhb.reshape(BH * P0 * SH, F_OUT), sp_lo, sp_hi, b_dec)
    out = jnp.concatenate([outa, outb], axis=0)
    return out.reshape(B, P0, F_OUT)
